# Initial kernel scaffold; baseline (speedup 1.0000x reference)
import jax
import jax.numpy as jnp
from jax.experimental import pallas as pl


def kernel(input_ids, time_table, space_table):
    raise NotImplementedError("write your pallas kernel here")


# speedup vs baseline: 1.0348x; 1.0348x over previous
import jax
import jax.numpy as jnp
from jax.experimental import pallas as pl

B, N, S, D = 8, 512, 256, 64
NB = 128


def _body(tt_ref, st_ref, out_ref):
    tt = tt_ref[...]
    st = st_ref[...]
    out_ref[0] = st[:, :, None] + tt[:, None, :]


def kernel(input_ids, time_table, space_table):
    del input_ids
    tt = time_table[:S].T
    st = space_table.T
    grid = (B, N // NB)
    return pl.pallas_call(
        _body,
        grid=grid,
        in_specs=[
            pl.BlockSpec((D, S), lambda b, j: (0, 0)),
            pl.BlockSpec((D, NB), lambda b, j: (0, j)),
        ],
        out_specs=pl.BlockSpec((1, D, NB, S), lambda b, j: (b, 0, j, 0)),
        out_shape=jax.ShapeDtypeStruct((B, D, N, S), jnp.float32),
    )(tt, st)
